# baseline (speedup 1.0000x reference)
import functools

import jax
import jax.numpy as jnp
from jax import lax
from jax.experimental import pallas as pl
from jax.experimental.pallas import tpu as pltpu
from jax.experimental.pallas import tpu_sc as plsc

BATCH = 16384
FIELDS = 26
EMBED = 32
VOCAB = 1000000
N_IDX = BATCH * FIELDS
LANES = 16

CHUNK = 128
NBUF = 8
TC_BN = 65536

V_SC = 327680
V_TC = VOCAB - V_SC
RS_W = 1024
RS_COLS_W = V_SC // 32
RS_NCHUNK = RS_COLS_W // RS_W
TC_OFF = V_SC // TC_BN


def _rowsum_tc(table_t):
    def body(t_ref, o_ref):
        o_ref[...] = jnp.sum(t_ref[...], axis=0)

    return pl.pallas_call(
        body,
        grid=(pl.cdiv(V_TC, TC_BN),),
        in_specs=[pl.BlockSpec((EMBED, TC_BN), lambda i: (0, i + TC_OFF))],
        out_specs=pl.BlockSpec((TC_BN,), lambda i: (i,)),
        out_shape=jax.ShapeDtypeStruct((V_TC,), jnp.float32),
    )(table_t)


def _make_rowsum_sc():
    mesh = plsc.VectorSubcoreMesh(core_axis_name="c", subcore_axis_name="s")
    nc = mesh.num_cores

    @functools.partial(
        pl.kernel,
        out_type=jax.ShapeDtypeStruct((V_SC,), jnp.float32),
        mesh=mesh,
        compiler_params=pltpu.CompilerParams(use_tc_tiling_on_sc=True),
        scratch_types=[
            pltpu.VMEM((EMBED, RS_W), jnp.float32),
            pltpu.VMEM((EMBED, RS_W), jnp.float32),
            pltpu.VMEM((RS_COLS_W,), jnp.float32),
            pltpu.SemaphoreType.DMA,
            pltpu.SemaphoreType.DMA,
        ],
    )
    def rs_kernel(table_hbm, out_hbm, slab0, slab1, outb, sem0, sem1):
        slabs = (slab0, slab1)
        sems = (sem0, sem1)
        wid = lax.axis_index("s") * nc + lax.axis_index("c")
        base = wid * RS_COLS_W

        def start(chunk, b):
            for t in range(EMBED // 8):
                pltpu.async_copy(
                    table_hbm.at[pl.ds(t * 8, 8),
                                 pl.ds(base + chunk * RS_W, RS_W)],
                    slabs[b].at[pl.ds(t * 8, 8), :],
                    sems[b])

        def wait(chunk, b):
            for t in range(EMBED // 8):
                pltpu.make_async_copy(
                    table_hbm.at[pl.ds(t * 8, 8),
                                 pl.ds(base + chunk * RS_W, RS_W)],
                    slabs[b].at[pl.ds(t * 8, 8), :],
                    sems[b]).wait()

        start(0, 0)
        start(1, 1)

        def outer(g, carry):
            for b in range(2):
                chunk = g * 2 + b
                wait(chunk, b)

                def inner(i, carry):
                    for u in range(4):
                        q = i * 4 + u
                        c = slabs[b][0, pl.ds(q * LANES, LANES)]
                        for r in range(1, EMBED):
                            c = c + slabs[b][r, pl.ds(q * LANES, LANES)]
                        outb[pl.ds(chunk * RS_W + q * LANES, LANES)] = c
                    return carry

                lax.fori_loop(0, RS_W // LANES // 4, inner, 0)
                nxt = chunk + 2

                @pl.when(nxt < RS_NCHUNK)
                def _():
                    start(nxt, b)
            return carry

        lax.fori_loop(0, RS_NCHUNK // 2, outer, 0)
        pltpu.sync_copy(outb, out_hbm.at[pl.ds(wid * RS_COLS_W, RS_COLS_W)])

    return rs_kernel


def _make_sc_kernel(nw):
    cols_w = BATCH // nw
    n_chunks = FIELDS * (cols_w // CHUNK)
    k_per_row = cols_w // CHUNK
    mesh = plsc.VectorSubcoreMesh(core_axis_name="c", subcore_axis_name="s")
    nc = mesh.num_cores

    @functools.partial(
        pl.kernel,
        out_type=jax.ShapeDtypeStruct((nw, LANES), jnp.float32),
        mesh=mesh,
        compiler_params=pltpu.CompilerParams(use_tc_tiling_on_sc=True),
        scratch_types=[
            pltpu.VMEM((FIELDS, cols_w), jnp.int32),
            pltpu.VMEM((CHUNK,), jnp.float32),
            pltpu.VMEM((CHUNK,), jnp.float32),
            pltpu.VMEM((CHUNK,), jnp.float32),
            pltpu.VMEM((CHUNK,), jnp.float32),
            pltpu.VMEM((CHUNK,), jnp.float32),
            pltpu.VMEM((CHUNK,), jnp.float32),
            pltpu.VMEM((CHUNK,), jnp.float32),
            pltpu.VMEM((CHUNK,), jnp.float32),
            pltpu.VMEM((LANES,), jnp.float32),
            pltpu.SemaphoreType.DMA,
            pltpu.SemaphoreType.DMA,
            pltpu.SemaphoreType.DMA,
            pltpu.SemaphoreType.DMA,
            pltpu.SemaphoreType.DMA,
            pltpu.SemaphoreType.DMA,
            pltpu.SemaphoreType.DMA,
            pltpu.SemaphoreType.DMA,
        ],
    )
    def sc_kernel(idx_hbm, rowsum_hbm, out_hbm,
                  idx_v, buf0, buf1, buf2, buf3, buf4, buf5, buf6, buf7,
                  outv, sem0, sem1, sem2, sem3, sem4, sem5, sem6, sem7):
        bufs = (buf0, buf1, buf2, buf3, buf4, buf5, buf6, buf7)
        sems = (sem0, sem1, sem2, sem3, sem4, sem5, sem6, sem7)
        wid = lax.axis_index("s") * nc + lax.axis_index("c")

        pltpu.sync_copy(
            idx_hbm.at[:, pl.ds(wid * cols_w, cols_w)], idx_v)

        def idx_slice(j):
            return idx_v.at[j // k_per_row,
                            pl.ds((j % k_per_row) * CHUNK, CHUNK)]

        for b in range(NBUF):
            pltpu.async_copy(rowsum_hbm.at[idx_slice(b)], bufs[b], sems[b])

        zero = jnp.zeros((LANES,), jnp.float32)

        def group(g, acc):
            for b in range(NBUF):
                j = g * NBUF + b
                pltpu.make_async_copy(
                    rowsum_hbm.at[idx_slice(j)], bufs[b], sems[b]).wait()
                c = bufs[b][0:LANES]
                for r in range(1, CHUNK // LANES):
                    c = c + bufs[b][r * LANES:(r + 1) * LANES]
                acc = acc + c
                nj = j + NBUF

                @pl.when(nj < n_chunks)
                def _():
                    pltpu.async_copy(
                        rowsum_hbm.at[idx_slice(nj)], bufs[b], sems[b])
            return acc

        acc = lax.fori_loop(0, n_chunks // NBUF, group, zero)
        outv[...] = acc
        pltpu.sync_copy(outv, out_hbm.at[wid])

    return sc_kernel


def kernel(x, table):
    nw = 32
    table_t = table.T
    rs_tc = _rowsum_tc(table_t)
    rs_sc = _make_rowsum_sc()(table_t)
    rowsum = jnp.concatenate([rs_sc, rs_tc])
    idx_t = x.astype(jnp.int32).T
    partials = _make_sc_kernel(nw)(idx_t, rowsum)
    return jnp.sum(partials) / jnp.float32(N_IDX * EMBED)
